# baseline (speedup 1.0000x reference)
import functools

import jax
import jax.numpy as jnp
from jax.experimental import pallas as pl

_N = 200
_D = 4
_E = 8
_NPID = 5
_PIDS_NZ = (1, 2, 3, 4)
_ALPHA = 32.0
_A2 = _ALPHA * _ALPHA
_NR = 208
_K = _D + _NPID + 3


def _chamfer_kernel(lxe_ref, ryt_ref, lye_ref, rxt_ref, ip_ref, op_ref,
                    nz_ref, z_ref, *, n_batches):
    i = pl.program_id(0)

    lxe = lxe_ref[...]
    ryt = ryt_ref[...]
    lye = lye_ref[...]
    rxt = rxt_ref[...]
    ip = ip_ref[...]
    op = op_ref[...]

    inv_b = jnp.float32(1.0 / n_batches)

    dn = (((2,), (1,)), ((0,), (0,)))
    out1 = jax.lax.dot_general(lxe, ryt, dn,
                               preferred_element_type=jnp.float32,
                               precision=jax.lax.Precision.HIGHEST)
    out2 = jax.lax.dot_general(lye, rxt, dn,
                               preferred_element_type=jnp.float32,
                               precision=jax.lax.Precision.HIGHEST)

    cmin2 = jnp.min(out1[:, :_N, :], axis=1)
    rmin2 = jnp.min(out2[:, :_N, :], axis=1)
    cmin = jnp.sqrt(jnp.maximum(cmin2, 0.0))
    rmin = jnp.sqrt(jnp.maximum(rmin2, 0.0))
    norm_y = jnp.sqrt(jnp.maximum(out1[:, _N, :] - 2.0 * _A2, 0.0))
    norm_x = jnp.sqrt(jnp.maximum(out2[:, _N, :] - 2.0 * _A2, 0.0))

    mz = op == 0
    n0 = jnp.maximum(1, jnp.sum(mz, axis=1, keepdims=True)).astype(jnp.float32)
    loss_zero = jnp.sum(jnp.where(mz, norm_y, 0.0), axis=1, keepdims=True) / n0
    z_ref[...] = loss_zero * inv_b

    loss_nz = jnp.zeros((_E, 1), jnp.float32)
    for p in _PIDS_NZ:
        mx = ip == p
        my = op == p
        nin = jnp.sum(mx, axis=1, keepdims=True)
        nout = jnp.sum(my, axis=1, keepdims=True)
        ninp = jnp.maximum(1, nin).astype(jnp.float32)
        noutp = jnp.maximum(1, nout).astype(jnp.float32)
        s_a = jnp.sum(jnp.where(mx, norm_x, 0.0), axis=1, keepdims=True)
        s_b = jnp.sum(jnp.where(my, norm_y, 0.0), axis=1, keepdims=True)
        s_cx = jnp.sum(jnp.where(mx, rmin, 0.0), axis=1, keepdims=True)
        s_cy = jnp.sum(jnp.where(my, cmin, 0.0), axis=1, keepdims=True)
        loss_a = s_a / ninp
        loss_b = s_b / noutp
        loss_c = 0.5 * (s_cx / noutp + s_cy / ninp)
        loss_p = jnp.where(nout == 0, loss_a, jnp.where(nin == 0, loss_b, loss_c))
        loss_nz = loss_nz + loss_p

    @pl.when(i == 0)
    def _():
        nz_ref[...] = jnp.zeros((1, 1), jnp.float32)

    nz_ref[...] += jnp.sum(loss_nz).reshape(1, 1) * inv_b


def _build_operands(pts, pid, n2):
    b = pts.shape[0]
    f32 = jnp.float32
    oh = jax.nn.one_hot(pid, _NPID, dtype=f32)
    ones = jnp.ones((b, _N, 1), f32)
    zeros = jnp.zeros((b, _N, 1), f32)
    n2a = (n2 + _A2)[:, :, None]
    lhs = jnp.concatenate(
        [-2.0 * pts, (-2.0 * _ALPHA) * oh, n2a, ones, zeros], axis=2)
    extract = (jnp.zeros((b, _NR - _N, _K), f32)
               .at[:, 0, _K - 1].set(1.0)
               .at[:, 0, _D + _NPID].set(2.0 * _A2)
               .at[:, 1:, _D + _NPID].set(1e9))
    lhs_ext = jnp.concatenate([lhs, extract], axis=1)
    rhs = jnp.concatenate(
        [pts, _ALPHA * oh, ones, n2a, n2[:, :, None]], axis=2)
    rhs_t = jnp.transpose(rhs, (0, 2, 1))
    return lhs_ext, rhs_t


def kernel(target, reco, in_pid, out_pid):
    b = target.shape[0]
    nx2 = jnp.sum(target * target, axis=2)
    ny2 = jnp.sum(reco * reco, axis=2)
    lxe, rxt = _build_operands(target, in_pid, nx2)
    lye, ryt = _build_operands(reco, out_pid, ny2)
    steps = b // _E

    nz, z = pl.pallas_call(
        functools.partial(_chamfer_kernel, n_batches=b),
        grid=(steps,),
        in_specs=[
            pl.BlockSpec((_E, _NR, _K), lambda i: (i, 0, 0)),
            pl.BlockSpec((_E, _K, _N), lambda i: (i, 0, 0)),
            pl.BlockSpec((_E, _NR, _K), lambda i: (i, 0, 0)),
            pl.BlockSpec((_E, _K, _N), lambda i: (i, 0, 0)),
            pl.BlockSpec((_E, _N), lambda i: (i, 0)),
            pl.BlockSpec((_E, _N), lambda i: (i, 0)),
        ],
        out_specs=[
            pl.BlockSpec((1, 1), lambda i: (0, 0)),
            pl.BlockSpec((_E, 1), lambda i: (i, 0)),
        ],
        out_shape=[
            jax.ShapeDtypeStruct((1, 1), jnp.float32),
            jax.ShapeDtypeStruct((b, 1), jnp.float32),
        ],
    )(lxe, ryt, lye, rxt, in_pid, out_pid)

    return nz.reshape(()), z.reshape(b)


# speedup vs baseline: 1.2364x; 1.2364x over previous
import functools

import jax
import jax.numpy as jnp
from jax.experimental import pallas as pl

_N = 200
_D = 4
_E = 8
_NPID = 5
_PIDS_NZ = (1, 2, 3, 4)
_ALPHA = 32.0
_A2 = _ALPHA * _ALPHA
_NR = 208
_K = _D + _NPID + 3


def _chamfer_kernel(lxe_ref, ryt_ref, lye_ref, rxt_ref, ip_ref, op_ref,
                    nz_ref, z_ref, *, n_batches):
    i = pl.program_id(0)

    lxe = lxe_ref[...]
    ryt = ryt_ref[...]
    lye = lye_ref[...]
    rxt = rxt_ref[...]
    ip = ip_ref[...]
    op = op_ref[...]

    inv_b = jnp.float32(1.0 / n_batches)

    dn = (((2,), (1,)), ((0,), (0,)))
    out1 = jax.lax.dot_general(lxe, ryt, dn,
                               preferred_element_type=jnp.float32)
    out2 = jax.lax.dot_general(lye, rxt, dn,
                               preferred_element_type=jnp.float32)

    cmin2 = jnp.min(out1[:, :_N, :], axis=1)
    rmin2 = jnp.min(out2[:, :_N, :], axis=1)
    cmin = jnp.sqrt(jnp.maximum(cmin2, 0.0))
    rmin = jnp.sqrt(jnp.maximum(rmin2, 0.0))
    norm_y = jnp.sqrt(jnp.maximum(out1[:, _N, :] - 2.0 * _A2, 0.0))
    norm_x = jnp.sqrt(jnp.maximum(out2[:, _N, :] - 2.0 * _A2, 0.0))

    mz = op == 0
    n0 = jnp.maximum(1, jnp.sum(mz, axis=1, keepdims=True)).astype(jnp.float32)
    loss_zero = jnp.sum(jnp.where(mz, norm_y, 0.0), axis=1, keepdims=True) / n0
    z_ref[...] = loss_zero * inv_b

    loss_nz = jnp.zeros((_E, 1), jnp.float32)
    for p in _PIDS_NZ:
        mx = ip == p
        my = op == p
        nin = jnp.sum(mx, axis=1, keepdims=True)
        nout = jnp.sum(my, axis=1, keepdims=True)
        ninp = jnp.maximum(1, nin).astype(jnp.float32)
        noutp = jnp.maximum(1, nout).astype(jnp.float32)
        s_a = jnp.sum(jnp.where(mx, norm_x, 0.0), axis=1, keepdims=True)
        s_b = jnp.sum(jnp.where(my, norm_y, 0.0), axis=1, keepdims=True)
        s_cx = jnp.sum(jnp.where(mx, rmin, 0.0), axis=1, keepdims=True)
        s_cy = jnp.sum(jnp.where(my, cmin, 0.0), axis=1, keepdims=True)
        loss_a = s_a / ninp
        loss_b = s_b / noutp
        loss_c = 0.5 * (s_cx / noutp + s_cy / ninp)
        loss_p = jnp.where(nout == 0, loss_a, jnp.where(nin == 0, loss_b, loss_c))
        loss_nz = loss_nz + loss_p

    @pl.when(i == 0)
    def _():
        nz_ref[...] = jnp.zeros((1, 1), jnp.float32)

    nz_ref[...] += jnp.sum(loss_nz).reshape(1, 1) * inv_b


def _build_operands(pts, pid, n2):
    b = pts.shape[0]
    f32 = jnp.float32
    oh = jax.nn.one_hot(pid, _NPID, dtype=f32)
    ones = jnp.ones((b, _N, 1), f32)
    zeros = jnp.zeros((b, _N, 1), f32)
    n2a = (n2 + _A2)[:, :, None]
    lhs = jnp.concatenate(
        [-2.0 * pts, (-2.0 * _ALPHA) * oh, n2a, ones, zeros], axis=2)
    extract = (jnp.zeros((b, _NR - _N, _K), f32)
               .at[:, 0, _K - 1].set(1.0)
               .at[:, 0, _D + _NPID].set(2.0 * _A2)
               .at[:, 1:, _D + _NPID].set(1e9))
    lhs_ext = jnp.concatenate([lhs, extract], axis=1)
    rhs = jnp.concatenate(
        [pts, _ALPHA * oh, ones, n2a, n2[:, :, None]], axis=2)
    rhs_t = jnp.transpose(rhs, (0, 2, 1))
    return lhs_ext, rhs_t


def kernel(target, reco, in_pid, out_pid):
    b = target.shape[0]
    nx2 = jnp.sum(target * target, axis=2)
    ny2 = jnp.sum(reco * reco, axis=2)
    lxe, rxt = _build_operands(target, in_pid, nx2)
    lye, ryt = _build_operands(reco, out_pid, ny2)
    steps = b // _E

    nz, z = pl.pallas_call(
        functools.partial(_chamfer_kernel, n_batches=b),
        grid=(steps,),
        in_specs=[
            pl.BlockSpec((_E, _NR, _K), lambda i: (i, 0, 0)),
            pl.BlockSpec((_E, _K, _N), lambda i: (i, 0, 0)),
            pl.BlockSpec((_E, _NR, _K), lambda i: (i, 0, 0)),
            pl.BlockSpec((_E, _K, _N), lambda i: (i, 0, 0)),
            pl.BlockSpec((_E, _N), lambda i: (i, 0)),
            pl.BlockSpec((_E, _N), lambda i: (i, 0)),
        ],
        out_specs=[
            pl.BlockSpec((1, 1), lambda i: (0, 0)),
            pl.BlockSpec((_E, 1), lambda i: (i, 0)),
        ],
        out_shape=[
            jax.ShapeDtypeStruct((1, 1), jnp.float32),
            jax.ShapeDtypeStruct((b, 1), jnp.float32),
        ],
    )(lxe, ryt, lye, rxt, in_pid, out_pid)

    return nz.reshape(()), z.reshape(b)


# speedup vs baseline: 2.3071x; 1.8660x over previous
import functools

import jax
import jax.numpy as jnp
from jax.experimental import pallas as pl

_N = 200
_D = 4
_E = 8
_PIDS_NZ = (1, 2, 3, 4)
_K = _D + 1
_BIG2 = 1e18


def _chamfer_kernel(lhs_ref, rht_ref, nx2_ref, ny2_ref, ip2_ref, op2_ref,
                    ip3_ref, op3_ref, nz_ref, z_ref, *, n_batches):
    i = pl.program_id(0)

    lhs = lhs_ref[...]
    rht = rht_ref[...]
    nx2 = nx2_ref[...]
    ny2 = ny2_ref[...]
    ip2 = ip2_ref[...]
    op2 = op2_ref[...]
    ip3 = ip3_ref[...]
    op3 = op3_ref[...]

    inv_b = jnp.float32(1.0 / n_batches)

    norm_x = jnp.sqrt(nx2)
    norm_y = jnp.sqrt(ny2)

    mz = op2 == 0
    n0 = jnp.maximum(1, jnp.sum(mz, axis=1, keepdims=True)).astype(jnp.float32)
    loss_zero = jnp.sum(jnp.where(mz, norm_y, 0.0), axis=1, keepdims=True) / n0
    z_ref[...] = loss_zero * inv_b

    dn = (((2,), (1,)), ((0,), (0,)))
    cross = jax.lax.dot_general(lhs, rht, dn,
                                preferred_element_type=jnp.float32)
    nx2_col = jnp.sum(lhs[:, :, :_D] * lhs[:, :, :_D], axis=2,
                      keepdims=True) * 0.25
    d2 = nx2_col + cross

    valid = ip3 == op3
    dm2 = jnp.where(valid, d2, jnp.float32(_BIG2))
    rmin = jnp.sqrt(jnp.maximum(jnp.min(dm2, axis=2), 0.0))
    cmin = jnp.sqrt(jnp.maximum(jnp.min(dm2, axis=1), 0.0))

    loss_nz = jnp.zeros((_E, 1), jnp.float32)
    for p in _PIDS_NZ:
        mx = ip2 == p
        my = op2 == p
        nin = jnp.sum(mx, axis=1, keepdims=True)
        nout = jnp.sum(my, axis=1, keepdims=True)
        ninp = jnp.maximum(1, nin).astype(jnp.float32)
        noutp = jnp.maximum(1, nout).astype(jnp.float32)
        s_a = jnp.sum(jnp.where(mx, norm_x, 0.0), axis=1, keepdims=True)
        s_b = jnp.sum(jnp.where(my, norm_y, 0.0), axis=1, keepdims=True)
        s_cx = jnp.sum(jnp.where(mx, rmin, 0.0), axis=1, keepdims=True)
        s_cy = jnp.sum(jnp.where(my, cmin, 0.0), axis=1, keepdims=True)
        loss_a = s_a / ninp
        loss_b = s_b / noutp
        loss_c = 0.5 * (s_cx / noutp + s_cy / ninp)
        loss_p = jnp.where(nout == 0, loss_a, jnp.where(nin == 0, loss_b, loss_c))
        loss_nz = loss_nz + loss_p

    @pl.when(i == 0)
    def _():
        nz_ref[...] = jnp.zeros((1, 1), jnp.float32)

    nz_ref[...] += jnp.sum(loss_nz).reshape(1, 1) * inv_b


def kernel(target, reco, in_pid, out_pid):
    b, n, d = target.shape
    f32 = jnp.float32
    nx2 = jnp.sum(target * target, axis=2)
    ny2 = jnp.sum(reco * reco, axis=2)
    ones = jnp.ones((b, n, 1), f32)
    lhs = jnp.concatenate([-2.0 * target, ones], axis=2)
    rhs = jnp.concatenate([reco, ny2[:, :, None]], axis=2)
    rht = jnp.transpose(rhs, (0, 2, 1))
    ip3 = in_pid.reshape(b, n, 1)
    op3 = out_pid.reshape(b, 1, n)
    steps = b // _E

    nz, z = pl.pallas_call(
        functools.partial(_chamfer_kernel, n_batches=b),
        grid=(steps,),
        in_specs=[
            pl.BlockSpec((_E, n, _K), lambda i: (i, 0, 0)),
            pl.BlockSpec((_E, _K, n), lambda i: (i, 0, 0)),
            pl.BlockSpec((_E, n), lambda i: (i, 0)),
            pl.BlockSpec((_E, n), lambda i: (i, 0)),
            pl.BlockSpec((_E, n), lambda i: (i, 0)),
            pl.BlockSpec((_E, n), lambda i: (i, 0)),
            pl.BlockSpec((_E, n, 1), lambda i: (i, 0, 0)),
            pl.BlockSpec((_E, 1, n), lambda i: (i, 0, 0)),
        ],
        out_specs=[
            pl.BlockSpec((1, 1), lambda i: (0, 0)),
            pl.BlockSpec((_E, 1), lambda i: (i, 0)),
        ],
        out_shape=[
            jax.ShapeDtypeStruct((1, 1), jnp.float32),
            jax.ShapeDtypeStruct((b, 1), jnp.float32),
        ],
    )(lhs, rht, nx2, ny2, in_pid, out_pid, ip3, op3)

    return nz.reshape(()), z.reshape(b)


# speedup vs baseline: 3.0488x; 1.3215x over previous
import functools

import jax
import jax.numpy as jnp
from jax.experimental import pallas as pl

_N = 200
_D = 4
_E = 16
_PIDS_NZ = (1, 2, 3, 4)
_BIG2 = 1e18


def _chamfer_kernel(x_ref, yt_ref, nx2_ref, ny2_ref, ip2_ref, op2_ref,
                    ip3_ref, op3_ref, nz_ref, z_ref, *, n_batches):
    i = pl.program_id(0)

    x = x_ref[...]
    yt = yt_ref[...]
    nx2 = nx2_ref[...]
    ny2 = ny2_ref[...]
    ip2 = ip2_ref[...]
    op2 = op2_ref[...]
    ip3 = ip3_ref[...]
    op3 = op3_ref[...]

    inv_b = jnp.float32(1.0 / n_batches)

    norm_x = jnp.sqrt(nx2)
    norm_y = jnp.sqrt(ny2)

    mz = op2 == 0
    n0 = jnp.maximum(1, jnp.sum(mz, axis=1, keepdims=True)).astype(jnp.float32)
    loss_zero = jnp.sum(jnp.where(mz, norm_y, 0.0), axis=1, keepdims=True) / n0
    z_ref[...] = loss_zero * inv_b

    xy = jax.lax.dot_general(
        x, yt, (((2,), (1,)), ((0,), (0,))),
        preferred_element_type=jnp.float32,
    )
    nx2_col = jnp.sum(x * x, axis=2, keepdims=True)
    d2 = nx2_col + ny2[:, None, :] - 2.0 * xy

    valid = ip3 == op3
    dm2 = jnp.where(valid, d2, jnp.float32(_BIG2))
    rmin = jnp.sqrt(jnp.maximum(jnp.min(dm2, axis=2), 0.0))
    cmin = jnp.sqrt(jnp.maximum(jnp.min(dm2, axis=1), 0.0))

    loss_nz = jnp.zeros((_E, 1), jnp.float32)
    for p in _PIDS_NZ:
        mx = ip2 == p
        my = op2 == p
        nin = jnp.sum(mx, axis=1, keepdims=True)
        nout = jnp.sum(my, axis=1, keepdims=True)
        ninp = jnp.maximum(1, nin).astype(jnp.float32)
        noutp = jnp.maximum(1, nout).astype(jnp.float32)
        s_a = jnp.sum(jnp.where(mx, norm_x, 0.0), axis=1, keepdims=True)
        s_b = jnp.sum(jnp.where(my, norm_y, 0.0), axis=1, keepdims=True)
        s_cx = jnp.sum(jnp.where(mx, rmin, 0.0), axis=1, keepdims=True)
        s_cy = jnp.sum(jnp.where(my, cmin, 0.0), axis=1, keepdims=True)
        loss_a = s_a / ninp
        loss_b = s_b / noutp
        loss_c = 0.5 * (s_cx / noutp + s_cy / ninp)
        loss_p = jnp.where(nout == 0, loss_a, jnp.where(nin == 0, loss_b, loss_c))
        loss_nz = loss_nz + loss_p

    @pl.when(i == 0)
    def _():
        nz_ref[...] = jnp.zeros((1, 1), jnp.float32)

    nz_ref[...] += jnp.sum(loss_nz).reshape(1, 1) * inv_b


def kernel(target, reco, in_pid, out_pid):
    b, n, d = target.shape
    nx2 = jnp.sum(target * target, axis=2)
    ny2 = jnp.sum(reco * reco, axis=2)
    yt = jnp.transpose(reco, (0, 2, 1))
    ip3 = in_pid.reshape(b, n, 1)
    op3 = out_pid.reshape(b, 1, n)
    steps = b // _E

    nz, z = pl.pallas_call(
        functools.partial(_chamfer_kernel, n_batches=b),
        grid=(steps,),
        in_specs=[
            pl.BlockSpec((_E, n, d), lambda i: (i, 0, 0)),
            pl.BlockSpec((_E, d, n), lambda i: (i, 0, 0)),
            pl.BlockSpec((_E, n), lambda i: (i, 0)),
            pl.BlockSpec((_E, n), lambda i: (i, 0)),
            pl.BlockSpec((_E, n), lambda i: (i, 0)),
            pl.BlockSpec((_E, n), lambda i: (i, 0)),
            pl.BlockSpec((_E, n, 1), lambda i: (i, 0, 0)),
            pl.BlockSpec((_E, 1, n), lambda i: (i, 0, 0)),
        ],
        out_specs=[
            pl.BlockSpec((1, 1), lambda i: (0, 0)),
            pl.BlockSpec((_E, 1), lambda i: (i, 0)),
        ],
        out_shape=[
            jax.ShapeDtypeStruct((1, 1), jnp.float32),
            jax.ShapeDtypeStruct((b, 1), jnp.float32),
        ],
    )(target, yt, nx2, ny2, in_pid, out_pid, ip3, op3)

    return nz.reshape(()), z.reshape(b)
